# baseline (speedup 1.0000x reference)
import functools

import jax
import jax.numpy as jnp
from jax import lax
from jax.experimental import pallas as pl
from jax.experimental.pallas import tpu as pltpu
from jax.experimental.pallas import tpu_sc as plsc

NC = 2
NS = 16
NW = NC * NS


def _sc_gather4(uidx, gidx, t_gu, t_gg, t_mu, t_mg):
    B = uidx.shape[0]
    D = t_gu.shape[1]
    bpw = B // NW
    assert B % (8 * NW) == 0

    mesh = plsc.VectorSubcoreMesh(
        core_axis_name="c", subcore_axis_name="s", num_cores=NC,
        num_subcores=NS)
    K = 16

    @functools.partial(
        pl.kernel,
        mesh=mesh,
        compiler_params=pltpu.CompilerParams(use_tc_tiling_on_sc=True),
        out_type=[jax.ShapeDtypeStruct((B, D), jnp.float32)] * 4,
        scratch_types=[
            pltpu.VMEM((bpw,), jnp.int32),
            pltpu.VMEM((bpw,), jnp.int32),
            pltpu.SemaphoreType.DMA,
        ],
    )
    def k(uidx_hbm, gidx_hbm, gu_hbm, gg_hbm, mu_hbm, mg_hbm,
          out_gu, out_gg, out_mu, out_mg,
          idxu_v, idxg_v, sem):
        wid = lax.axis_index("s") * NC + lax.axis_index("c")
        base = wid * bpw
        rows = pl.ds(base, bpw)
        pltpu.sync_copy(uidx_hbm.at[rows], idxu_v)
        pltpu.sync_copy(gidx_hbm.at[rows], idxg_v)

        @pl.loop(0, bpw, step=K)
        def _(r0):
            vu = idxu_v[pl.ds(r0, K)]
            vg = idxg_v[pl.ds(r0, K)]
            cps = []
            for j in range(K):
                iu = vu[j]
                ig = vg[j]
                dst = pl.ds(base + r0 + j, 1)
                cps.append(pltpu.async_copy(
                    gu_hbm.at[pl.ds(iu, 1)], out_gu.at[dst], sem))
                cps.append(pltpu.async_copy(
                    mu_hbm.at[pl.ds(iu, 1)], out_mu.at[dst], sem))
                cps.append(pltpu.async_copy(
                    gg_hbm.at[pl.ds(ig, 1)], out_gg.at[dst], sem))
                cps.append(pltpu.async_copy(
                    mg_hbm.at[pl.ds(ig, 1)], out_mg.at[dst], sem))
            for cp in cps:
                cp.wait()

    return k(uidx, gidx, t_gu, t_gg, t_mu, t_mg)


def _tc_dense(gu, gg, mu, mg, w1u, w1g, b1, w2, b2, w3, b3, wg, wm, bfc):
    B, D = gu.shape
    blk = 2048

    def body(gu_r, gg_r, mu_r, mg_r, w1u_r, w1g_r, b1_r, w2_r, b2_r,
             w3_r, b3_r, wg_r, wm_r, bfc_r, out_r):
        f32 = jnp.float32
        gcf = jnp.maximum(gu_r[...] * gg_r[...], 0.0)
        h = jnp.dot(mu_r[...], w1u_r[...], preferred_element_type=f32)
        h = h + jnp.dot(mg_r[...], w1g_r[...], preferred_element_type=f32)
        h = jnp.maximum(h + b1_r[...], 0.0)
        h = jnp.maximum(
            jnp.dot(h, w2_r[...], preferred_element_type=f32) + b2_r[...], 0.0)
        h = jnp.maximum(
            jnp.dot(h, w3_r[...], preferred_element_type=f32) + b3_r[...], 0.0)
        out_r[...] = (jnp.dot(gcf, wg_r[...], preferred_element_type=f32)
                      + jnp.dot(h, wm_r[...], preferred_element_type=f32)
                      + bfc_r[...])

    row_spec = pl.BlockSpec((blk, D), lambda i: (i, 0))
    full = lambda a: pl.BlockSpec(a.shape, lambda i: (0,) * a.ndim)
    return pl.pallas_call(
        body,
        grid=(B // blk,),
        in_specs=[row_spec, row_spec, row_spec, row_spec,
                  full(w1u), full(w1g), full(b1), full(w2), full(b2),
                  full(w3), full(b3), full(wg), full(wm), full(bfc)],
        out_specs=pl.BlockSpec((blk, 1), lambda i: (i, 0)),
        out_shape=jax.ShapeDtypeStruct((B, 1), jnp.float32),
    )(gu, gg, mu, mg, w1u, w1g, b1, w2, b2, w3, b3, wg, wm, bfc)


def kernel(user_index, game_index, emb_gcf_user, emb_gcf_game, emb_mlp_user,
           emb_mlp_game, W1, b1, W2, b2, W3, b3, Wfc, bfc):
    D = emb_gcf_user.shape[1]
    uidx = user_index.astype(jnp.int32)
    gidx = game_index.astype(jnp.int32)
    gu, gg, mu, mg = _sc_gather4(uidx, gidx, emb_gcf_user, emb_gcf_game,
                                 emb_mlp_user, emb_mlp_game)
    w1u = W1[:, :D].T
    w1g = W1[:, D:].T
    wg = Wfc[:, :D].T
    wm = Wfc[:, D:].T
    out = _tc_dense(gu, gg, mu, mg, w1u, w1g, b1.reshape(1, -1),
                    W2.T, b2.reshape(1, -1), W3.T, b3.reshape(1, -1),
                    wg, wm, bfc.reshape(1, 1))
    return out


# speedup vs baseline: 2.1468x; 2.1468x over previous
import functools

import jax
import jax.numpy as jnp
from jax import lax
from jax.experimental import pallas as pl
from jax.experimental.pallas import tpu as pltpu
from jax.experimental.pallas import tpu_sc as plsc

NC = 2
NS = 16
NW = NC * NS


def _sc_gather4(uidx, gidx, t_gu, t_gg, t_mu, t_mg):
    B = uidx.shape[0]
    D = t_gu.shape[1]
    bpw = B // NW
    assert B % (8 * NW) == 0

    mesh = plsc.VectorSubcoreMesh(
        core_axis_name="c", subcore_axis_name="s", num_cores=NC,
        num_subcores=NS)
    K = 16
    C = 128

    @functools.partial(
        pl.kernel,
        mesh=mesh,
        compiler_params=pltpu.CompilerParams(use_tc_tiling_on_sc=True),
        out_type=[jax.ShapeDtypeStruct((B, D), jnp.float32)] * 4,
        scratch_types=[
            pltpu.VMEM((bpw,), jnp.int32),
            pltpu.VMEM((bpw,), jnp.int32),
            pltpu.VMEM((C, D), jnp.float32),
            pltpu.VMEM((C, D), jnp.float32),
            pltpu.VMEM((C, D), jnp.float32),
            pltpu.VMEM((C, D), jnp.float32),
            pltpu.SemaphoreType.DMA,
        ],
    )
    def k(uidx_hbm, gidx_hbm, gu_hbm, gg_hbm, mu_hbm, mg_hbm,
          out_gu, out_gg, out_mu, out_mg,
          idxu_v, idxg_v, bgu, bmu, bgg, bmg, sem):
        wid = lax.axis_index("s") * NC + lax.axis_index("c")
        base = wid * bpw
        pltpu.sync_copy(uidx_hbm.at[pl.ds(base, bpw)], idxu_v)
        pltpu.sync_copy(gidx_hbm.at[pl.ds(base, bpw)], idxg_v)

        @pl.loop(0, bpw, step=C)
        def _(c0):
            @pl.loop(0, C, step=K)
            def _(r0):
                vu = idxu_v[pl.ds(c0 + r0, K)]
                vg = idxg_v[pl.ds(c0 + r0, K)]
                cps = []
                for j in range(K):
                    iu = vu[j]
                    ig = vg[j]
                    dst = pl.ds(r0 + j, 1)
                    cps.append(pltpu.async_copy(
                        gu_hbm.at[pl.ds(iu, 1)], bgu.at[dst], sem))
                    cps.append(pltpu.async_copy(
                        mu_hbm.at[pl.ds(iu, 1)], bmu.at[dst], sem))
                    cps.append(pltpu.async_copy(
                        gg_hbm.at[pl.ds(ig, 1)], bgg.at[dst], sem))
                    cps.append(pltpu.async_copy(
                        mg_hbm.at[pl.ds(ig, 1)], bmg.at[dst], sem))
                for cp in cps:
                    cp.wait()

            rows = pl.ds(base + c0, C)
            pltpu.sync_copy(bgu, out_gu.at[rows])
            pltpu.sync_copy(bmu, out_mu.at[rows])
            pltpu.sync_copy(bgg, out_gg.at[rows])
            pltpu.sync_copy(bmg, out_mg.at[rows])

    return k(uidx, gidx, t_gu, t_gg, t_mu, t_mg)


def _tc_dense(gu, gg, mu, mg, w1u, w1g, b1, w2, b2, w3, b3, wg, wm, bfc):
    B, D = gu.shape
    blk = 2048

    def body(gu_r, gg_r, mu_r, mg_r, w1u_r, w1g_r, b1_r, w2_r, b2_r,
             w3_r, b3_r, wg_r, wm_r, bfc_r, out_r):
        f32 = jnp.float32
        gcf = jnp.maximum(gu_r[...] * gg_r[...], 0.0)
        h = jnp.dot(mu_r[...], w1u_r[...], preferred_element_type=f32)
        h = h + jnp.dot(mg_r[...], w1g_r[...], preferred_element_type=f32)
        h = jnp.maximum(h + b1_r[...], 0.0)
        h = jnp.maximum(
            jnp.dot(h, w2_r[...], preferred_element_type=f32) + b2_r[...], 0.0)
        h = jnp.maximum(
            jnp.dot(h, w3_r[...], preferred_element_type=f32) + b3_r[...], 0.0)
        out_r[...] = (jnp.dot(gcf, wg_r[...], preferred_element_type=f32)
                      + jnp.dot(h, wm_r[...], preferred_element_type=f32)
                      + bfc_r[...])

    row_spec = pl.BlockSpec((blk, D), lambda i: (i, 0))
    full = lambda a: pl.BlockSpec(a.shape, lambda i: (0,) * a.ndim)
    return pl.pallas_call(
        body,
        grid=(B // blk,),
        in_specs=[row_spec, row_spec, row_spec, row_spec,
                  full(w1u), full(w1g), full(b1), full(w2), full(b2),
                  full(w3), full(b3), full(wg), full(wm), full(bfc)],
        out_specs=pl.BlockSpec((blk, 1), lambda i: (i, 0)),
        out_shape=jax.ShapeDtypeStruct((B, 1), jnp.float32),
    )(gu, gg, mu, mg, w1u, w1g, b1, w2, b2, w3, b3, wg, wm, bfc)


def kernel(user_index, game_index, emb_gcf_user, emb_gcf_game, emb_mlp_user,
           emb_mlp_game, W1, b1, W2, b2, W3, b3, Wfc, bfc):
    D = emb_gcf_user.shape[1]
    uidx = user_index.astype(jnp.int32)
    gidx = game_index.astype(jnp.int32)
    gu, gg, mu, mg = _sc_gather4(uidx, gidx, emb_gcf_user, emb_gcf_game,
                                 emb_mlp_user, emb_mlp_game)
    w1u = W1[:, :D].T
    w1g = W1[:, D:].T
    wg = Wfc[:, :D].T
    wm = Wfc[:, D:].T
    out = _tc_dense(gu, gg, mu, mg, w1u, w1g, b1.reshape(1, -1),
                    W2.T, b2.reshape(1, -1), W3.T, b3.reshape(1, -1),
                    wg, wm, bfc.reshape(1, 1))
    return out
